# baseline (speedup 1.0000x reference)
import functools

import jax
import jax.numpy as jnp
from jax.experimental import pallas as pl
from jax.experimental.pallas import tpu as pltpu

_N = 10000
_E = 160000
_D = 256
_ROWS = 1000


def _mm_body(do_relu, agg_ref, x_ref, wrel_ref, wroot_ref, b_ref, o_ref):
    h = (
        jnp.dot(agg_ref[...], wrel_ref[...], preferred_element_type=jnp.float32)
        + jnp.dot(x_ref[...], wroot_ref[...], preferred_element_type=jnp.float32)
        + b_ref[...]
    )
    if do_relu:
        h = jnp.maximum(h, 0.0)
    o_ref[...] = h


def _mm(agg, x, wrel, wroot, b, do_relu):
    n = agg.shape[0]
    grid = n // _ROWS
    return pl.pallas_call(
        functools.partial(_mm_body, do_relu),
        grid=(grid,),
        in_specs=[
            pl.BlockSpec((_ROWS, _D), lambda i: (i, 0)),
            pl.BlockSpec((_ROWS, _D), lambda i: (i, 0)),
            pl.BlockSpec((_D, _D), lambda i: (0, 0)),
            pl.BlockSpec((_D, _D), lambda i: (0, 0)),
            pl.BlockSpec((1, _D), lambda i: (0, 0)),
        ],
        out_specs=pl.BlockSpec((_ROWS, _D), lambda i: (i, 0)),
        out_shape=jax.ShapeDtypeStruct((n, _D), jnp.float32),
    )(agg, x, wrel, wroot, b.reshape(1, _D))


def kernel(x, edge_index, W1_rel, W1_root, b1, W2_rel, W2_root, b2):
    src = edge_index[0]
    dst = edge_index[1]
    agg = jax.ops.segment_sum(jnp.take(x, src, axis=0), dst, num_segments=_N)
    t = _mm(agg, x, W1_rel, W1_root, b1, do_relu=True)
    agg2 = jax.ops.segment_sum(jnp.take(t, src, axis=0), dst, num_segments=_N)
    out = _mm(agg2, t, W2_rel, W2_root, b2, do_relu=False)
    return out


# speedup vs baseline: 5.7579x; 5.7579x over previous
import functools

import jax
import jax.numpy as jnp
from jax import lax
from jax.experimental import pallas as pl
from jax.experimental.pallas import tpu as pltpu
from jax.experimental.pallas import tpu_sc as plsc

_N = 10000
_E = 160000
_D = 256
_HALF = 128

_NS = 16
_CHUNK = 128
_KPW = 80
_EPAD = _NS * _KPW * _CHUNK
_ACC_ROWS = 10112
_ZROWS = _ACC_ROWS // _NS
_OROWS = 632
_OLAST = _N - 15 * _OROWS

_ROWS = 1000


_mesh = plsc.VectorSubcoreMesh(core_axis_name="c", subcore_axis_name="s")


@functools.partial(
    pl.kernel,
    out_type=[
        jax.ShapeDtypeStruct((_N, _HALF), jnp.float32),
        jax.ShapeDtypeStruct((_N, _HALF), jnp.float32),
    ],
    mesh=_mesh,
    scratch_types=[
        pltpu.VMEM_SHARED((_ACC_ROWS, _HALF), jnp.float32),
        pltpu.VMEM((_KPW, _CHUNK), jnp.int32),
        pltpu.VMEM((_KPW, _CHUNK), jnp.int32),
        pltpu.VMEM((_CHUNK, _HALF), jnp.float32),
        pltpu.SemaphoreType.DMA,
    ],
)
def _segsum(x0, x1, src2d, dst2d, zeros_hbm, out0, out1,
            acc, src_v, dst_v, rows_v, sem):
    c = lax.axis_index("c")
    s = lax.axis_index("s")
    pltpu.sync_copy(zeros_hbm.at[pl.ds(s * _ZROWS, _ZROWS)],
                    acc.at[pl.ds(s * _ZROWS, _ZROWS)])
    pltpu.sync_copy(src2d.at[pl.ds(s * _KPW, _KPW)], src_v)
    pltpu.sync_copy(dst2d.at[pl.ds(s * _KPW, _KPW)], dst_v)
    plsc.subcore_barrier()

    def body(g, carry):
        @pl.when(c == 0)
        def _():
            pltpu.async_copy(x0.at[src_v.at[g]], rows_v, sem).wait()

        @pl.when(c == 1)
        def _():
            pltpu.async_copy(x1.at[src_v.at[g]], rows_v, sem).wait()

        pltpu.sync_copy(rows_v, acc.at[dst_v.at[g]], add=True)
        return carry

    lax.fori_loop(0, _KPW, body, 0)
    plsc.subcore_barrier()

    @pl.when((c == 0) & (s < 15))
    def _():
        pltpu.sync_copy(acc.at[pl.ds(s * _OROWS, _OROWS)],
                        out0.at[pl.ds(s * _OROWS, _OROWS)])

    @pl.when((c == 0) & (s == 15))
    def _():
        pltpu.sync_copy(acc.at[pl.ds(15 * _OROWS, _OLAST)],
                        out0.at[pl.ds(15 * _OROWS, _OLAST)])

    @pl.when((c == 1) & (s < 15))
    def _():
        pltpu.sync_copy(acc.at[pl.ds(s * _OROWS, _OROWS)],
                        out1.at[pl.ds(s * _OROWS, _OROWS)])

    @pl.when((c == 1) & (s == 15))
    def _():
        pltpu.sync_copy(acc.at[pl.ds(15 * _OROWS, _OLAST)],
                        out1.at[pl.ds(15 * _OROWS, _OLAST)])


def _mm_body(do_relu, split_out, a0, a1, r0, r1, wrel, wroot, b, *outs):
    h = (
        jnp.dot(a0[...], wrel[0:_HALF, :], preferred_element_type=jnp.float32)
        + jnp.dot(a1[...], wrel[_HALF:_D, :], preferred_element_type=jnp.float32)
        + jnp.dot(r0[...], wroot[0:_HALF, :], preferred_element_type=jnp.float32)
        + jnp.dot(r1[...], wroot[_HALF:_D, :], preferred_element_type=jnp.float32)
        + b[...]
    )
    if do_relu:
        h = jnp.maximum(h, 0.0)
    if split_out:
        outs[0][...] = h[:, 0:_HALF]
        outs[1][...] = h[:, _HALF:_D]
    else:
        outs[0][...] = h


def _mm(a0, a1, r0, r1, wrel, wroot, b, do_relu, split_out):
    grid = _N // _ROWS
    half_spec = pl.BlockSpec((_ROWS, _HALF), lambda i: (i, 0))
    w_spec = pl.BlockSpec((_D, _D), lambda i: (0, 0))
    if split_out:
        out_specs = [half_spec, half_spec]
        out_shape = [jax.ShapeDtypeStruct((_N, _HALF), jnp.float32)] * 2
    else:
        out_specs = pl.BlockSpec((_ROWS, _D), lambda i: (i, 0))
        out_shape = jax.ShapeDtypeStruct((_N, _D), jnp.float32)
    return pl.pallas_call(
        functools.partial(_mm_body, do_relu, split_out),
        grid=(grid,),
        in_specs=[half_spec, half_spec, half_spec, half_spec,
                  w_spec, w_spec, pl.BlockSpec((1, _D), lambda i: (0, 0))],
        out_specs=out_specs,
        out_shape=out_shape,
    )(a0, a1, r0, r1, wrel, wroot, b.reshape(1, _D))


def kernel(x, edge_index, W1_rel, W1_root, b1, W2_rel, W2_root, b2):
    src = edge_index[0]
    dst = edge_index[1]
    npad = _EPAD - _E
    pad_ids = jnp.arange(npad, dtype=jnp.int32)
    src_p = jnp.concatenate([src, pad_ids % _N])
    dst_p = jnp.concatenate([dst, _N + (pad_ids % (_ACC_ROWS - _N))])
    src2d = src_p.reshape(_EPAD // _CHUNK, _CHUNK)
    dst2d = dst_p.reshape(_EPAD // _CHUNK, _CHUNK)
    zeros = jnp.zeros((_ACC_ROWS, _HALF), jnp.float32)

    x0 = x[:, :_HALF]
    x1 = x[:, _HALF:]
    a0, a1 = _segsum(x0, x1, src2d, dst2d, zeros)
    t0, t1 = _mm(a0, a1, x0, x1, W1_rel, W1_root, b1,
                 do_relu=True, split_out=True)
    a0, a1 = _segsum(t0, t1, src2d, dst2d, zeros)
    out = _mm(a0, a1, t0, t1, W2_rel, W2_root, b2,
              do_relu=False, split_out=False)
    return out


# speedup vs baseline: 8.3336x; 1.4473x over previous
import functools

import jax
import jax.numpy as jnp
from jax import lax
from jax.experimental import pallas as pl
from jax.experimental.pallas import tpu as pltpu
from jax.experimental.pallas import tpu_sc as plsc

_N = 10000
_E = 160000
_D = 256
_HALF = 128

_NS = 16
_CHUNK = 128
_KPW = 80
_EPAD = _NS * _KPW * _CHUNK
_ACC_ROWS = 10112
_ZROWS = _ACC_ROWS // _NS
_OROWS = 632
_OLAST = _N - 15 * _OROWS
_GRP = 40

_ROWS = 1000


_mesh = plsc.VectorSubcoreMesh(core_axis_name="c", subcore_axis_name="s")


@functools.partial(
    pl.kernel,
    out_type=[
        jax.ShapeDtypeStruct((_N, _HALF), jnp.float32),
        jax.ShapeDtypeStruct((_N, _HALF), jnp.float32),
    ],
    mesh=_mesh,
    scratch_types=[
        pltpu.VMEM_SHARED((_ACC_ROWS, _HALF), jnp.float32),
        pltpu.VMEM((_GRP, _CHUNK), jnp.int32),
        pltpu.VMEM((_GRP, _CHUNK), jnp.int32),
        pltpu.VMEM((_CHUNK, _HALF), jnp.float32),
        pltpu.VMEM((_CHUNK, _HALF), jnp.float32),
        pltpu.SemaphoreType.DMA,
        pltpu.SemaphoreType.DMA,
    ],
)
def _segsum(x0, x1, src2d, dst2d, zeros_hbm, out0, out1,
            acc, src_v, dst_v, rows_a, rows_b, sem_a, sem_b):
    c = lax.axis_index("c")
    s = lax.axis_index("s")
    pltpu.sync_copy(zeros_hbm.at[pl.ds(s * _ZROWS, _ZROWS)],
                    acc.at[pl.ds(s * _ZROWS, _ZROWS)])
    plsc.subcore_barrier()

    def start_gather(g, buf, sem):
        @pl.when(c == 0)
        def _():
            pltpu.async_copy(x0.at[src_v.at[g]], buf, sem)

        @pl.when(c == 1)
        def _():
            pltpu.async_copy(x1.at[src_v.at[g]], buf, sem)

    def wait_gather(g, buf, sem):
        pltpu.make_async_copy(x0.at[src_v.at[g]], buf, sem).wait()

    def body(i, carry):
        g0 = 2 * i
        g1 = g0 + 1
        start_gather(g1, rows_b, sem_b)
        wait_gather(g0, rows_a, sem_a)
        pltpu.sync_copy(rows_a, acc.at[dst_v.at[g0]], add=True)

        @pl.when(g0 + 2 < _GRP)
        def _():
            start_gather(g0 + 2, rows_a, sem_a)

        wait_gather(g1, rows_b, sem_b)
        pltpu.sync_copy(rows_b, acc.at[dst_v.at[g1]], add=True)
        return carry

    for h in range(_KPW // _GRP):
        base = s * _KPW + h * _GRP
        pltpu.sync_copy(src2d.at[pl.ds(base, _GRP)], src_v)
        pltpu.sync_copy(dst2d.at[pl.ds(base, _GRP)], dst_v)
        start_gather(0, rows_a, sem_a)
        lax.fori_loop(0, _GRP // 2, body, 0)

    plsc.subcore_barrier()

    @pl.when((c == 0) & (s < 15))
    def _():
        pltpu.sync_copy(acc.at[pl.ds(s * _OROWS, _OROWS)],
                        out0.at[pl.ds(s * _OROWS, _OROWS)])

    @pl.when((c == 0) & (s == 15))
    def _():
        pltpu.sync_copy(acc.at[pl.ds(15 * _OROWS, _OLAST)],
                        out0.at[pl.ds(15 * _OROWS, _OLAST)])

    @pl.when((c == 1) & (s < 15))
    def _():
        pltpu.sync_copy(acc.at[pl.ds(s * _OROWS, _OROWS)],
                        out1.at[pl.ds(s * _OROWS, _OROWS)])

    @pl.when((c == 1) & (s == 15))
    def _():
        pltpu.sync_copy(acc.at[pl.ds(15 * _OROWS, _OLAST)],
                        out1.at[pl.ds(15 * _OROWS, _OLAST)])


def _mm_body(do_relu, split_out, a0, a1, r0, r1, wrel, wroot, b, *outs):
    h = (
        jnp.dot(a0[...], wrel[0:_HALF, :], preferred_element_type=jnp.float32)
        + jnp.dot(a1[...], wrel[_HALF:_D, :], preferred_element_type=jnp.float32)
        + jnp.dot(r0[...], wroot[0:_HALF, :], preferred_element_type=jnp.float32)
        + jnp.dot(r1[...], wroot[_HALF:_D, :], preferred_element_type=jnp.float32)
        + b[...]
    )
    if do_relu:
        h = jnp.maximum(h, 0.0)
    if split_out:
        outs[0][...] = h[:, 0:_HALF]
        outs[1][...] = h[:, _HALF:_D]
    else:
        outs[0][...] = h


def _mm(a0, a1, r0, r1, wrel, wroot, b, do_relu, split_out):
    grid = _N // _ROWS
    half_spec = pl.BlockSpec((_ROWS, _HALF), lambda i: (i, 0))
    w_spec = pl.BlockSpec((_D, _D), lambda i: (0, 0))
    if split_out:
        out_specs = [half_spec, half_spec]
        out_shape = [jax.ShapeDtypeStruct((_N, _HALF), jnp.float32)] * 2
    else:
        out_specs = pl.BlockSpec((_ROWS, _D), lambda i: (i, 0))
        out_shape = jax.ShapeDtypeStruct((_N, _D), jnp.float32)
    return pl.pallas_call(
        functools.partial(_mm_body, do_relu, split_out),
        grid=(grid,),
        in_specs=[half_spec, half_spec, half_spec, half_spec,
                  w_spec, w_spec, pl.BlockSpec((1, _D), lambda i: (0, 0))],
        out_specs=out_specs,
        out_shape=out_shape,
    )(a0, a1, r0, r1, wrel, wroot, b.reshape(1, _D))


def kernel(x, edge_index, W1_rel, W1_root, b1, W2_rel, W2_root, b2):
    src = edge_index[0]
    dst = edge_index[1]
    npad = _EPAD - _E
    pad_ids = jnp.arange(npad, dtype=jnp.int32)
    src_p = jnp.concatenate([src, pad_ids % _N])
    dst_p = jnp.concatenate([dst, _N + (pad_ids % (_ACC_ROWS - _N))])
    src2d = src_p.reshape(_EPAD // _CHUNK, _CHUNK)
    dst2d = dst_p.reshape(_EPAD // _CHUNK, _CHUNK)
    zeros = jnp.zeros((_ACC_ROWS, _HALF), jnp.float32)

    x0 = x[:, :_HALF]
    x1 = x[:, _HALF:]
    a0, a1 = _segsum(x0, x1, src2d, dst2d, zeros)
    t0, t1 = _mm(a0, a1, x0, x1, W1_rel, W1_root, b1,
                 do_relu=True, split_out=True)
    a0, a1 = _segsum(t0, t1, src2d, dst2d, zeros)
    out = _mm(a0, a1, t0, t1, W2_rel, W2_root, b2,
              do_relu=False, split_out=False)
    return out


# speedup vs baseline: 8.4499x; 1.0140x over previous
import functools

import jax
import jax.numpy as jnp
from jax import lax
from jax.experimental import pallas as pl
from jax.experimental.pallas import tpu as pltpu
from jax.experimental.pallas import tpu_sc as plsc

_N = 10000
_E = 160000
_D = 256
_HALF = 128

_NS = 16
_CHUNK = 128
_KPW = 80
_EPAD = _NS * _KPW * _CHUNK
_ACC_ROWS = 10112
_ZROWS = _ACC_ROWS // _NS
_OROWS = 632
_OLAST = _N - 15 * _OROWS
_GRP = 40

_ROWS = 1000


_mesh = plsc.VectorSubcoreMesh(core_axis_name="c", subcore_axis_name="s")


@functools.partial(
    pl.kernel,
    out_type=[
        jax.ShapeDtypeStruct((_N, _HALF), jnp.float32),
        jax.ShapeDtypeStruct((_N, _HALF), jnp.float32),
    ],
    mesh=_mesh,
    scratch_types=[
        pltpu.VMEM_SHARED((_ACC_ROWS, _HALF), jnp.float32),
        pltpu.VMEM((_GRP, _CHUNK), jnp.int32),
        pltpu.VMEM((_GRP, _CHUNK), jnp.int32),
        pltpu.VMEM((_CHUNK, _HALF), jnp.float32),
        pltpu.VMEM((_CHUNK, _HALF), jnp.float32),
        pltpu.SemaphoreType.DMA,
        pltpu.SemaphoreType.DMA,
    ],
)
def _segsum(x0, x1, src2d, dst2d, zeros_hbm, out0, out1,
            acc, src_v, dst_v, rows_a, rows_b, sem_a, sem_b):
    c = lax.axis_index("c")
    s = lax.axis_index("s")
    pltpu.sync_copy(zeros_hbm.at[pl.ds(s * _ZROWS, _ZROWS)],
                    acc.at[pl.ds(s * _ZROWS, _ZROWS)])
    plsc.subcore_barrier()

    def start_gather(g, buf, sem):
        @pl.when(c == 0)
        def _():
            pltpu.async_copy(x0.at[src_v.at[g]], buf, sem)

        @pl.when(c == 1)
        def _():
            pltpu.async_copy(x1.at[src_v.at[g]], buf, sem)

    def wait_gather(g, buf, sem):
        pltpu.make_async_copy(x0.at[src_v.at[g]], buf, sem).wait()

    def body(i, carry):
        g0 = 2 * i
        g1 = g0 + 1
        start_gather(g1, rows_b, sem_b)
        wait_gather(g0, rows_a, sem_a)
        pltpu.sync_copy(rows_a, acc.at[dst_v.at[g0]], add=True)

        @pl.when(g0 + 2 < _GRP)
        def _():
            start_gather(g0 + 2, rows_a, sem_a)

        wait_gather(g1, rows_b, sem_b)
        pltpu.sync_copy(rows_b, acc.at[dst_v.at[g1]], add=True)
        return carry

    for h in range(_KPW // _GRP):
        base = s * _KPW + h * _GRP
        pltpu.sync_copy(src2d.at[pl.ds(base, _GRP)], src_v)
        pltpu.sync_copy(dst2d.at[pl.ds(base, _GRP)], dst_v)
        start_gather(0, rows_a, sem_a)
        lax.fori_loop(0, _GRP // 2, body, 0)

    plsc.subcore_barrier()

    @pl.when((c == 0) & (s < 15))
    def _():
        pltpu.sync_copy(acc.at[pl.ds(s * _OROWS, _OROWS)],
                        out0.at[pl.ds(s * _OROWS, _OROWS)])

    @pl.when((c == 0) & (s == 15))
    def _():
        pltpu.sync_copy(acc.at[pl.ds(15 * _OROWS, _OLAST)],
                        out0.at[pl.ds(15 * _OROWS, _OLAST)])

    @pl.when((c == 1) & (s < 15))
    def _():
        pltpu.sync_copy(acc.at[pl.ds(s * _OROWS, _OROWS)],
                        out1.at[pl.ds(s * _OROWS, _OROWS)])

    @pl.when((c == 1) & (s == 15))
    def _():
        pltpu.sync_copy(acc.at[pl.ds(15 * _OROWS, _OLAST)],
                        out1.at[pl.ds(15 * _OROWS, _OLAST)])


def _mm_root_body(r0, r1, wroot, b, o):
    o[...] = (
        jnp.dot(r0[...], wroot[0:_HALF, :], preferred_element_type=jnp.float32)
        + jnp.dot(r1[...], wroot[_HALF:_D, :], preferred_element_type=jnp.float32)
        + b[...]
    )


def _mm_root(r0, r1, wroot, b):
    half_spec = pl.BlockSpec((_ROWS, _HALF), lambda i: (i, 0))
    w_spec = pl.BlockSpec((_D, _D), lambda i: (0, 0))
    return pl.pallas_call(
        _mm_root_body,
        grid=(_N // _ROWS,),
        in_specs=[half_spec, half_spec, w_spec,
                  pl.BlockSpec((1, _D), lambda i: (0, 0))],
        out_specs=pl.BlockSpec((_ROWS, _D), lambda i: (i, 0)),
        out_shape=jax.ShapeDtypeStruct((_N, _D), jnp.float32),
    )(r0, r1, wroot, b.reshape(1, _D))


def _mm_rel_body(do_relu, split_out, a0, a1, wrel, root, *outs):
    h = (
        jnp.dot(a0[...], wrel[0:_HALF, :], preferred_element_type=jnp.float32)
        + jnp.dot(a1[...], wrel[_HALF:_D, :], preferred_element_type=jnp.float32)
        + root[...]
    )
    if do_relu:
        h = jnp.maximum(h, 0.0)
    if split_out:
        outs[0][...] = h[:, 0:_HALF]
        outs[1][...] = h[:, _HALF:_D]
    else:
        outs[0][...] = h


def _mm_rel(a0, a1, wrel, root, do_relu, split_out):
    half_spec = pl.BlockSpec((_ROWS, _HALF), lambda i: (i, 0))
    full_spec = pl.BlockSpec((_ROWS, _D), lambda i: (i, 0))
    w_spec = pl.BlockSpec((_D, _D), lambda i: (0, 0))
    if split_out:
        out_specs = [half_spec, half_spec]
        out_shape = [jax.ShapeDtypeStruct((_N, _HALF), jnp.float32)] * 2
    else:
        out_specs = full_spec
        out_shape = jax.ShapeDtypeStruct((_N, _D), jnp.float32)
    return pl.pallas_call(
        functools.partial(_mm_rel_body, do_relu, split_out),
        grid=(_N // _ROWS,),
        in_specs=[half_spec, half_spec, w_spec, full_spec],
        out_specs=out_specs,
        out_shape=out_shape,
    )(a0, a1, wrel, root)


def kernel(x, edge_index, W1_rel, W1_root, b1, W2_rel, W2_root, b2):
    src = edge_index[0]
    dst = edge_index[1]
    npad = _EPAD - _E
    pad_ids = jnp.arange(npad, dtype=jnp.int32)
    src_p = jnp.concatenate([src, pad_ids % _N])
    dst_p = jnp.concatenate([dst, _N + (pad_ids % (_ACC_ROWS - _N))])
    src2d = src_p.reshape(_EPAD // _CHUNK, _CHUNK)
    dst2d = dst_p.reshape(_EPAD // _CHUNK, _CHUNK)
    zeros = jnp.zeros((_ACC_ROWS, _HALF), jnp.float32)

    x0 = x[:, :_HALF]
    x1 = x[:, _HALF:]
    a0, a1 = _segsum(x0, x1, src2d, dst2d, zeros)
    root1 = _mm_root(x0, x1, W1_root, b1)
    t0, t1 = _mm_rel(a0, a1, W1_rel, root1, do_relu=True, split_out=True)
    a0, a1 = _segsum(t0, t1, src2d, dst2d, zeros)
    root2 = _mm_root(t0, t1, W2_root, b2)
    out = _mm_rel(a0, a1, W2_rel, root2, do_relu=False, split_out=False)
    return out


# speedup vs baseline: 8.4669x; 1.0020x over previous
import functools

import numpy as np

import jax
import jax.numpy as jnp
from jax import lax
from jax.experimental import pallas as pl
from jax.experimental.pallas import tpu as pltpu
from jax.experimental.pallas import tpu_sc as plsc

_N = 10000
_E = 160000
_D = 256
_HALF = 128

_NS = 16
_CHUNK = 128
_KPW = 80
_EPAD = _NS * _KPW * _CHUNK
_ACC_ROWS = 10112
_ZROWS = _ACC_ROWS // _NS
_OROWS = 632
_OLAST = _N - 15 * _OROWS
_GRP = 40

_ROWS = 1000

_PAD_SRC = (np.arange(_EPAD - _E) % _N).astype(np.int32)
_PAD_DST = (_N + np.arange(_EPAD - _E) % (_ACC_ROWS - _N)).astype(np.int32)
_ZEROS = np.zeros((_ACC_ROWS, _HALF), np.float32)


_mesh = plsc.VectorSubcoreMesh(core_axis_name="c", subcore_axis_name="s")


@functools.partial(
    pl.kernel,
    out_type=[
        jax.ShapeDtypeStruct((_N, _HALF), jnp.float32),
        jax.ShapeDtypeStruct((_N, _HALF), jnp.float32),
    ],
    mesh=_mesh,
    scratch_types=[
        pltpu.VMEM_SHARED((_ACC_ROWS, _HALF), jnp.float32),
        pltpu.VMEM((_GRP, _CHUNK), jnp.int32),
        pltpu.VMEM((_GRP, _CHUNK), jnp.int32),
        pltpu.VMEM((_CHUNK, _HALF), jnp.float32),
        pltpu.VMEM((_CHUNK, _HALF), jnp.float32),
        pltpu.SemaphoreType.DMA,
        pltpu.SemaphoreType.DMA,
    ],
)
def _segsum(x0, x1, src2d, dst2d, zeros_hbm, out0, out1,
            acc, src_v, dst_v, rows_a, rows_b, sem_a, sem_b):
    c = lax.axis_index("c")
    s = lax.axis_index("s")
    pltpu.sync_copy(zeros_hbm.at[pl.ds(s * _ZROWS, _ZROWS)],
                    acc.at[pl.ds(s * _ZROWS, _ZROWS)])
    plsc.subcore_barrier()

    def start_gather(g, buf, sem):
        @pl.when(c == 0)
        def _():
            pltpu.async_copy(x0.at[src_v.at[g]], buf, sem)

        @pl.when(c == 1)
        def _():
            pltpu.async_copy(x1.at[src_v.at[g]], buf, sem)

    def wait_gather(g, buf, sem):
        pltpu.make_async_copy(x0.at[src_v.at[g]], buf, sem).wait()

    def body(i, carry):
        g0 = 2 * i
        g1 = g0 + 1
        start_gather(g1, rows_b, sem_b)
        wait_gather(g0, rows_a, sem_a)
        pltpu.sync_copy(rows_a, acc.at[dst_v.at[g0]], add=True)

        @pl.when(g0 + 2 < _GRP)
        def _():
            start_gather(g0 + 2, rows_a, sem_a)

        wait_gather(g1, rows_b, sem_b)
        pltpu.sync_copy(rows_b, acc.at[dst_v.at[g1]], add=True)
        return carry

    for h in range(_KPW // _GRP):
        base = s * _KPW + h * _GRP
        pltpu.sync_copy(src2d.at[pl.ds(base, _GRP)], src_v)
        pltpu.sync_copy(dst2d.at[pl.ds(base, _GRP)], dst_v)
        start_gather(0, rows_a, sem_a)
        lax.fori_loop(0, _GRP // 2, body, 0)

    plsc.subcore_barrier()

    @pl.when((c == 0) & (s < 15))
    def _():
        pltpu.sync_copy(acc.at[pl.ds(s * _OROWS, _OROWS)],
                        out0.at[pl.ds(s * _OROWS, _OROWS)])

    @pl.when((c == 0) & (s == 15))
    def _():
        pltpu.sync_copy(acc.at[pl.ds(15 * _OROWS, _OLAST)],
                        out0.at[pl.ds(15 * _OROWS, _OLAST)])

    @pl.when((c == 1) & (s < 15))
    def _():
        pltpu.sync_copy(acc.at[pl.ds(s * _OROWS, _OROWS)],
                        out1.at[pl.ds(s * _OROWS, _OROWS)])

    @pl.when((c == 1) & (s == 15))
    def _():
        pltpu.sync_copy(acc.at[pl.ds(15 * _OROWS, _OLAST)],
                        out1.at[pl.ds(15 * _OROWS, _OLAST)])


def _mm_root_body(r0, r1, wroot, b, o):
    o[...] = (
        jnp.dot(r0[...], wroot[0:_HALF, :], preferred_element_type=jnp.float32)
        + jnp.dot(r1[...], wroot[_HALF:_D, :], preferred_element_type=jnp.float32)
        + b[...]
    )


def _mm_root(r0, r1, wroot, b):
    half_spec = pl.BlockSpec((_ROWS, _HALF), lambda i: (i, 0))
    w_spec = pl.BlockSpec((_D, _D), lambda i: (0, 0))
    return pl.pallas_call(
        _mm_root_body,
        grid=(_N // _ROWS,),
        in_specs=[half_spec, half_spec, w_spec,
                  pl.BlockSpec((1, _D), lambda i: (0, 0))],
        out_specs=pl.BlockSpec((_ROWS, _D), lambda i: (i, 0)),
        out_shape=jax.ShapeDtypeStruct((_N, _D), jnp.float32),
    )(r0, r1, wroot, b.reshape(1, _D))


def _mm_rel_body(do_relu, split_out, a0, a1, wrel, root, *outs):
    h = (
        jnp.dot(a0[...], wrel[0:_HALF, :], preferred_element_type=jnp.float32)
        + jnp.dot(a1[...], wrel[_HALF:_D, :], preferred_element_type=jnp.float32)
        + root[...]
    )
    if do_relu:
        h = jnp.maximum(h, 0.0)
    if split_out:
        outs[0][...] = h[:, 0:_HALF]
        outs[1][...] = h[:, _HALF:_D]
    else:
        outs[0][...] = h


def _mm_rel(a0, a1, wrel, root, do_relu, split_out):
    half_spec = pl.BlockSpec((_ROWS, _HALF), lambda i: (i, 0))
    full_spec = pl.BlockSpec((_ROWS, _D), lambda i: (i, 0))
    w_spec = pl.BlockSpec((_D, _D), lambda i: (0, 0))
    if split_out:
        out_specs = [half_spec, half_spec]
        out_shape = [jax.ShapeDtypeStruct((_N, _HALF), jnp.float32)] * 2
    else:
        out_specs = full_spec
        out_shape = jax.ShapeDtypeStruct((_N, _D), jnp.float32)
    return pl.pallas_call(
        functools.partial(_mm_rel_body, do_relu, split_out),
        grid=(_N // _ROWS,),
        in_specs=[half_spec, half_spec, w_spec, full_spec],
        out_specs=out_specs,
        out_shape=out_shape,
    )(a0, a1, wrel, root)


def kernel(x, edge_index, W1_rel, W1_root, b1, W2_rel, W2_root, b2):
    src = edge_index[0]
    dst = edge_index[1]
    src_p = jnp.concatenate([src, jnp.asarray(_PAD_SRC)])
    dst_p = jnp.concatenate([dst, jnp.asarray(_PAD_DST)])
    src2d = src_p.reshape(_EPAD // _CHUNK, _CHUNK)
    dst2d = dst_p.reshape(_EPAD // _CHUNK, _CHUNK)
    zeros = jnp.asarray(_ZEROS)

    x0 = x[:, :_HALF]
    x1 = x[:, _HALF:]
    a0, a1 = _segsum(x0, x1, src2d, dst2d, zeros)
    root1 = _mm_root(x0, x1, W1_root, b1)
    t0, t1 = _mm_rel(a0, a1, W1_rel, root1, do_relu=True, split_out=True)
    a0, a1 = _segsum(t0, t1, src2d, dst2d, zeros)
    root2 = _mm_root(t0, t1, W2_root, b2)
    out = _mm_rel(a0, a1, W2_rel, root2, do_relu=False, split_out=False)
    return out
